# Initial kernel scaffold; baseline (speedup 1.0000x reference)
import jax
import jax.numpy as jnp
from jax import lax
from jax.experimental import pallas as pl
from jax.experimental.pallas import tpu as pltpu
from jax.experimental.pallas import tpu_sc as plsc

_N = 10000
_E = 320000
_D = 128
_K = 10
_NC = 2
_NS = 16
_L = 16
_DH = _D // _NC
_ET = _E // _NS
_C = 128
_NCH = -(-_ET // _C)
_ETP = _NCH * _C
_NP = 10240
_BS = _NP // _NS
_RB = 128
_NB = _BS // _RB
_Q = _DH // _L


def _rsqrt16(x):
    i = plsc.bitcast(x, jnp.int32)
    i = jnp.int32(0x5F3759DF) - (i >> 1)
    y = plsc.bitcast(i, jnp.float32)
    for _ in range(3):
        y = y * (1.5 - 0.5 * x * y * y)
    return y


def _body(x_hbm, w_hbm, temp_hbm, src_hbm, dst_hbm, out_hbm,
          src_v, dst_v, nrm_v, bufA, bufB, zbuf, invo_v, invi_v,
          temp_v, dtmp,
          emb_sh, acc_sh, hid_sh, dego_sh, degi_sh):
    c = lax.axis_index("c")
    s = lax.axis_index("s")
    base = s * _BS
    z16 = jnp.zeros((_L,), jnp.float32)

    pltpu.sync_copy(src_hbm.at[s], src_v)
    pltpu.sync_copy(dst_hbm.at[s], dst_v)
    pltpu.sync_copy(w_hbm.at[s], nrm_v)
    pltpu.sync_copy(temp_hbm, temp_v)

    def _zb(r, _):
        for q in range(_Q):
            zbuf[r, pl.ds(q * _L, _L)] = z16
        return 0
    lax.fori_loop(0, _RB, _zb, 0)

    def _zd(i, _):
        dtmp[pl.ds(i * _L, _L)] = z16
        return 0
    lax.fori_loop(0, _BS // _L, _zd, 0)
    pltpu.sync_copy(dtmp, dego_sh.at[pl.ds(base, _BS)])
    pltpu.sync_copy(dtmp, degi_sh.at[pl.ds(base, _BS)])
    plsc.subcore_barrier()

    def _deg(j, _):
        pltpu.sync_copy(nrm_v.at[j], dego_sh.at[src_v.at[j]], add=True)
        pltpu.sync_copy(nrm_v.at[j], degi_sh.at[dst_v.at[j]], add=True)
        return 0
    lax.fori_loop(0, _NCH, _deg, 0)
    plsc.subcore_barrier()

    for dref in (dego_sh, degi_sh):
        pltpu.sync_copy(dref.at[pl.ds(base, _BS)], dtmp)

        def _inv(q, _):
            sl = pl.ds(q * _L, _L)
            d = dtmp[sl]
            dtmp[sl] = jnp.where(d > 0.0, _rsqrt16(jnp.maximum(d, 1e-30)), 0.0)
            return 0
        lax.fori_loop(0, _BS // _L, _inv, 0)
        pltpu.sync_copy(dtmp, dref.at[pl.ds(base, _BS)])
    plsc.subcore_barrier()
    pltpu.sync_copy(dego_sh, invo_v)
    pltpu.sync_copy(degi_sh, invi_v)

    def _nrm(j, _):
        for q in range(_C // _L):
            sl = pl.ds(q * _L, _L)
            sv = src_v[j, sl]
            dv = dst_v[j, sl]
            wv = nrm_v[j, sl]
            io = plsc.load_gather(invo_v, [sv])
            ii = plsc.load_gather(invi_v, [dv])
            nrm_v[j, sl] = wv * io * ii
        return 0
    lax.fori_loop(0, _NCH, _nrm, 0)

    t0 = temp_v[0]
    for b in range(_NB):
        r0 = base + b * _RB
        pltpu.sync_copy(x_hbm.at[c, pl.ds(r0, _RB)], bufA)
        pltpu.sync_copy(bufA, emb_sh.at[pl.ds(r0, _RB)])

        def _h0(r, _):
            for q in range(_Q):
                sl = pl.ds(q * _L, _L)
                bufB[r, sl] = bufA[r, sl] * t0
            return 0
        lax.fori_loop(0, _RB, _h0, 0)
        pltpu.sync_copy(bufB, hid_sh.at[pl.ds(r0, _RB)])
        pltpu.sync_copy(zbuf, acc_sh.at[pl.ds(r0, _RB)])
    plsc.subcore_barrier()

    for k in range(_K):
        tk = temp_v[k + 1]

        def _chunk(j, _):
            pltpu.sync_copy(emb_sh.at[src_v.at[j]], bufA)

            def _edge(e, _2):
                bv = jnp.full((_L,), nrm_v[j, e], jnp.float32)
                for q in range(_Q):
                    sl = pl.ds(q * _L, _L)
                    bufA[e, sl] = bufA[e, sl] * bv
                return 0
            lax.fori_loop(0, _C, _edge, 0)
            pltpu.sync_copy(bufA, acc_sh.at[dst_v.at[j]], add=True)
            return 0
        lax.fori_loop(0, _NCH, _chunk, 0)
        plsc.subcore_barrier()

        for b in range(_NB):
            r0 = base + b * _RB
            pltpu.sync_copy(acc_sh.at[pl.ds(r0, _RB)], bufA)
            pltpu.sync_copy(hid_sh.at[pl.ds(r0, _RB)], bufB)

            def _up(r, _):
                for q in range(_Q):
                    sl = pl.ds(q * _L, _L)
                    bufB[r, sl] = bufB[r, sl] + tk * bufA[r, sl]
                return 0
            lax.fori_loop(0, _RB, _up, 0)
            pltpu.sync_copy(bufB, hid_sh.at[pl.ds(r0, _RB)])
            pltpu.sync_copy(bufA, emb_sh.at[pl.ds(r0, _RB)])
            pltpu.sync_copy(zbuf, acc_sh.at[pl.ds(r0, _RB)])
        plsc.subcore_barrier()

    for b in range(_NB):
        r0 = base + b * _RB
        pltpu.sync_copy(hid_sh.at[pl.ds(r0, _RB)], bufA)
        pltpu.sync_copy(bufA, out_hbm.at[c, pl.ds(r0, _RB)])


def _make_call():
    mesh = plsc.VectorSubcoreMesh(core_axis_name="c", subcore_axis_name="s")
    return pl.kernel(
        _body,
        out_type=jax.ShapeDtypeStruct((_NC, _NP, _DH), jnp.float32),
        mesh=mesh,
        scratch_types=[
            pltpu.VMEM((_NCH, _C), jnp.int32),
            pltpu.VMEM((_NCH, _C), jnp.int32),
            pltpu.VMEM((_NCH, _C), jnp.float32),
            pltpu.VMEM((_RB, _DH), jnp.float32),
            pltpu.VMEM((_RB, _DH), jnp.float32),
            pltpu.VMEM((_RB, _DH), jnp.float32),
            pltpu.VMEM((_NP,), jnp.float32),
            pltpu.VMEM((_NP,), jnp.float32),
            pltpu.VMEM((_L,), jnp.float32),
            pltpu.VMEM((_BS,), jnp.float32),
            pltpu.VMEM_SHARED((_NP, _DH), jnp.float32),
            pltpu.VMEM_SHARED((_NP, _DH), jnp.float32),
            pltpu.VMEM_SHARED((_NP, _DH), jnp.float32),
            pltpu.VMEM_SHARED((_NP,), jnp.float32),
            pltpu.VMEM_SHARED((_NP,), jnp.float32),
        ],
    )


@jax.jit
def kernel(x, edge_weight, temp, edge_index):
    pad = _ETP - _ET
    src = jnp.pad(edge_index[0].reshape(_NS, _ET), ((0, 0), (0, pad)))
    dst = jnp.pad(edge_index[1].reshape(_NS, _ET), ((0, 0), (0, pad)))
    w = jnp.pad(edge_weight.reshape(_NS, _ET), ((0, 0), (0, pad)))
    src = src.reshape(_NS, _NCH, _C)
    dst = dst.reshape(_NS, _NCH, _C)
    w = w.reshape(_NS, _NCH, _C)
    xs = x.reshape(_N, _NC, _DH).transpose(1, 0, 2)
    xs = jnp.pad(xs, ((0, 0), (0, _NP - _N), (0, 0)))
    tp = jnp.pad(temp, (0, _L - (_K + 1)))
    out = _make_call()(xs, w, tp, src, dst)
    return out[:, :_N, :].transpose(1, 0, 2).reshape(_N, _D)


# speedup vs baseline: 1.3943x; 1.3943x over previous
import jax
import jax.numpy as jnp
from jax import lax
from jax.experimental import pallas as pl
from jax.experimental.pallas import tpu as pltpu
from jax.experimental.pallas import tpu_sc as plsc

_N = 10000
_E = 320000
_D = 128
_K = 10
_NS = 16
_L = 16
_ET = _E // _NS
_C = 64
_NCH = -(-_ET // _C)
_ETP = _NCH * _C
_NP = 10240
_BS = _NP // _NS
_RB = 64
_NB = _BS // _RB
_Q = _D // _L


def _rsqrt16(x):
    i = lax.bitcast_convert_type(x, jnp.int32)
    i = jnp.int32(0x5F3759DF) - (i >> 1)
    y = lax.bitcast_convert_type(i, jnp.float32)
    for _ in range(3):
        y = y * (1.5 - 0.5 * x * y * y)
    return y


def _body(x_hbm, w_hbm, temp_hbm, src_hbm, dst_hbm,
          out_hbm, emb_hbm, invo_hbm, invi_hbm, nrm_hbm,
          sidx, didx, nbuf, nrmv, bufA, bufB, temp_v, sem,
          acc_sh):
    s = lax.axis_index("s")
    base = s * _BS
    z16 = jnp.zeros((_L,), jnp.float32)

    pltpu.sync_copy(temp_hbm, temp_v)
    tv = temp_v[pl.ds(0, _L)]
    t0 = tv[0]
    lanes = lax.iota(jnp.int32, _L)

    def _zero_buf(buf):
        def _z(r, _):
            for q in range(_Q):
                buf[r, pl.ds(q * _L, _L)] = z16
            return 0
        lax.fori_loop(0, _RB, _z, 0)

    _zero_buf(bufA)
    for b in range(_NB):
        pltpu.sync_copy(bufA, acc_sh.at[pl.ds(base + b * _RB, _RB)])
    plsc.subcore_barrier()

    for idx_v, idx_hbm, inv_hbm in ((sidx, src_hbm, invo_hbm),
                                    (didx, dst_hbm, invi_hbm)):
        def _deg(j, _):
            pltpu.sync_copy(idx_hbm.at[s, j], idx_v)
            pltpu.sync_copy(w_hbm.at[s, j], nbuf)

            def _rep(g, _2):
                nv = nbuf[pl.ds(g * _L, _L)]
                for e16 in range(_L):
                    e = g * _L + e16
                    bv = jnp.full((_L,), nv[e16], jnp.float32)
                    for q in range(_Q):
                        bufA[e, pl.ds(q * _L, _L)] = bv
                return 0
            lax.fori_loop(0, _C // _L, _rep, 0)
            pltpu.async_copy(bufA, acc_sh.at[idx_v], sem, add=True).wait()
            return 0
        lax.fori_loop(0, _NCH, _deg, 0)
        plsc.subcore_barrier()

        _zero_buf(bufB)
        for b in range(_NB):
            r0 = base + b * _RB
            pltpu.sync_copy(acc_sh.at[pl.ds(r0, _RB)], bufA)

            def _inv(r, _):
                for q in range(_Q):
                    sl = pl.ds(q * _L, _L)
                    d = bufA[r, sl]
                    bufA[r, sl] = jnp.where(
                        d > 0.0, _rsqrt16(jnp.maximum(d, 1e-30)), 0.0)
                return 0
            lax.fori_loop(0, _RB, _inv, 0)
            pltpu.sync_copy(bufA, inv_hbm.at[pl.ds(r0, _RB)])
            pltpu.sync_copy(bufB, acc_sh.at[pl.ds(r0, _RB)])
        plsc.subcore_barrier()

    def _nrm(j, _):
        pltpu.sync_copy(src_hbm.at[s, j], sidx)
        pltpu.sync_copy(dst_hbm.at[s, j], didx)
        pltpu.sync_copy(w_hbm.at[s, j], nbuf)
        pltpu.async_copy(invo_hbm.at[sidx], bufA, sem).wait()
        pltpu.async_copy(invi_hbm.at[didx], bufB, sem).wait()

        def _ne(g, _2):
            nv = nbuf[pl.ds(g * _L, _L)]
            for e16 in range(_L):
                e = g * _L + e16
                bv = jnp.full((_L,), nv[e16], jnp.float32)
                nrmv[e, pl.ds(0, _L)] = (
                    bufA[e, pl.ds(0, _L)] * bufB[e, pl.ds(0, _L)] * bv)
            return 0
        lax.fori_loop(0, _C // _L, _ne, 0)
        pltpu.sync_copy(nrmv, nrm_hbm.at[s, j])
        return 0
    lax.fori_loop(0, _NCH, _nrm, 0)

    for b in range(_NB):
        r0 = base + b * _RB
        pltpu.sync_copy(x_hbm.at[pl.ds(r0, _RB)], bufA)
        pltpu.sync_copy(bufA, emb_hbm.at[pl.ds(r0, _RB)])

        def _h0(r, _):
            for q in range(_Q):
                sl = pl.ds(q * _L, _L)
                bufB[r, sl] = bufA[r, sl] * t0
            return 0
        lax.fori_loop(0, _RB, _h0, 0)
        pltpu.sync_copy(bufB, out_hbm.at[pl.ds(r0, _RB)])
    plsc.subcore_barrier()

    def _round(k, _):
        tk = lax.reduce_sum_p.bind(
            jnp.where(lanes == k + 1, tv, 0.0), axes=(0,))

        def _chunk(j, _1):
            pltpu.sync_copy(src_hbm.at[s, j], sidx)
            pltpu.async_copy(emb_hbm.at[sidx], bufA, sem).wait()
            pltpu.sync_copy(nrm_hbm.at[s, j], nrmv)
            pltpu.sync_copy(dst_hbm.at[s, j], didx)

            def _grp(e, _2):
                bv = nrmv[e, pl.ds(0, _L)]
                for q in range(_Q):
                    sl = pl.ds(q * _L, _L)
                    bufA[e, sl] = bufA[e, sl] * bv
                return 0
            lax.fori_loop(0, _C, _grp, 0)
            pltpu.async_copy(bufA, acc_sh.at[didx], sem, add=True).wait()
            return 0
        lax.fori_loop(0, _NCH, _chunk, 0)
        plsc.subcore_barrier()

        for b in range(_NB):
            r0 = base + b * _RB
            pltpu.sync_copy(acc_sh.at[pl.ds(r0, _RB)], bufA)
            pltpu.sync_copy(out_hbm.at[pl.ds(r0, _RB)], bufB)

            def _up(r, _2):
                for q in range(_Q):
                    sl = pl.ds(q * _L, _L)
                    bufB[r, sl] = bufB[r, sl] + tk * bufA[r, sl]
                return 0
            lax.fori_loop(0, _RB, _up, 0)
            pltpu.sync_copy(bufB, out_hbm.at[pl.ds(r0, _RB)])
            pltpu.sync_copy(bufA, emb_hbm.at[pl.ds(r0, _RB)])

            def _zz(r, _2):
                for q in range(_Q):
                    bufA[r, pl.ds(q * _L, _L)] = z16
                return 0
            lax.fori_loop(0, _RB, _zz, 0)
            pltpu.sync_copy(bufA, acc_sh.at[pl.ds(r0, _RB)])
        plsc.subcore_barrier()
        return 0
    lax.fori_loop(0, _K, _round, 0)


def _make_call():
    mesh = plsc.VectorSubcoreMesh(
        core_axis_name="c", subcore_axis_name="s", num_cores=1)
    return pl.kernel(
        _body,
        out_type=(
            jax.ShapeDtypeStruct((_NP, _D), jnp.float32),
            jax.ShapeDtypeStruct((_NP, _D), jnp.float32),
            jax.ShapeDtypeStruct((_NP, _D), jnp.float32),
            jax.ShapeDtypeStruct((_NP, _D), jnp.float32),
            jax.ShapeDtypeStruct((_NS, _NCH, _C, _L), jnp.float32),
        ),
        mesh=mesh,
        compiler_params=pltpu.CompilerParams(needs_layout_passes=False),
        scratch_types=[
            pltpu.VMEM((_C,), jnp.int32),
            pltpu.VMEM((_C,), jnp.int32),
            pltpu.VMEM((_C,), jnp.float32),
            pltpu.VMEM((_C, _L), jnp.float32),
            pltpu.VMEM((_RB, _D), jnp.float32),
            pltpu.VMEM((_RB, _D), jnp.float32),
            pltpu.VMEM((_L,), jnp.float32),
            pltpu.SemaphoreType.DMA,
            pltpu.VMEM_SHARED((_NP, _D), jnp.float32),
        ],
    )


@jax.jit
def kernel(x, edge_weight, temp, edge_index):
    pad = _ETP - _ET
    src = jnp.pad(edge_index[0].reshape(_NS, _ET), ((0, 0), (0, pad)))
    dst = jnp.pad(edge_index[1].reshape(_NS, _ET), ((0, 0), (0, pad)))
    w = jnp.pad(edge_weight.reshape(_NS, _ET), ((0, 0), (0, pad)))
    src = src.reshape(_NS, _NCH, _C)
    dst = dst.reshape(_NS, _NCH, _C)
    w = w.reshape(_NS, _NCH, _C)
    xs = jnp.pad(x, ((0, _NP - _N), (0, 0)))
    tp = jnp.pad(temp, (0, _L - (_K + 1)))
    out = _make_call()(xs, w, tp, src, dst)[0]
    return out[:_N]
